# baseline (speedup 1.0000x reference)
import functools

import jax
import jax.numpy as jnp
from jax import lax
from jax.experimental import pallas as pl
from jax.experimental.pallas import tpu as pltpu
from jax.experimental.pallas import tpu_sc as plsc

NC = 2
NS = 16
L = 16
CH = 80


def _elu(x):
  return jnp.where(x > 0, x, jnp.exp(x) - 1.0)


def _bcast(grp, j):
  v = jnp.sum(jnp.where(jnp.arange(L, dtype=jnp.int32) == j, grp, 0.0))
  return jnp.broadcast_to(v, (L,))


def _sc_conv1(tabs2, asad, src, dst, z2d, n, e):
  ept = e // NS
  nck = ept // CH
  nck2 = nck // 2
  assert nck2 * 2 == nck
  rpt = (n // NS) // 8 * 8
  tail = n - rpt * NS

  mesh = plsc.VectorSubcoreMesh(core_axis_name="c", subcore_axis_name="s",
                                num_cores=NC, num_subcores=NS)

  @functools.partial(
      pl.kernel,
      mesh=mesh,
      compiler_params=pltpu.CompilerParams(needs_layout_passes=False),
      out_type=(
          jax.ShapeDtypeStruct((2 * n, 128), jnp.float32),
          jax.ShapeDtypeStruct((2 * n,), jnp.float32),
          jax.ShapeDtypeStruct((e,), jnp.float32),
      ),
      scratch_types=dict(
          asn_v=pltpu.VMEM((n,), jnp.float32),
          adn_v=pltpu.VMEM((n,), jnp.float32),
          srcb=pltpu.VMEM((2, CH), jnp.int32),
          dstb=pltpu.VMEM((2, CH), jnp.int32),
          srcp=pltpu.VMEM((2, CH), jnp.int32),
          dsc=pltpu.VMEM((2, CH), jnp.int32),
          exb=pltpu.VMEM((2, CH), jnp.float32),
          rows=pltpu.VMEM((2, CH, 128), jnp.float32),
          sbuf=pltpu.VMEM((rpt,), jnp.float32),
          acc=pltpu.VMEM_SHARED((n, 128), jnp.float32),
          s_sh=pltpu.VMEM_SHARED((n,), jnp.float32),
          si0=pltpu.SemaphoreType.DMA, si1=pltpu.SemaphoreType.DMA,
          sj0=pltpu.SemaphoreType.DMA, sj1=pltpu.SemaphoreType.DMA,
          sg0=pltpu.SemaphoreType.DMA, sg1=pltpu.SemaphoreType.DMA,
          sr0=pltpu.SemaphoreType.DMA, sr1=pltpu.SemaphoreType.DMA,
          ss0=pltpu.SemaphoreType.DMA, ss1=pltpu.SemaphoreType.DMA,
          se0=pltpu.SemaphoreType.DMA, se1=pltpu.SemaphoreType.DMA,
      ),
  )
  def k(tabs_h, asad_h, src_h, dst_h, z2d_h,
        agg_o, s_o, ex_o,
        asn_v, adn_v, srcb, dstb, srcp, dsc, exb, rows, sbuf, acc, s_sh,
        si0, si1, sj0, sj1, sg0, sg1, sr0, sr1, ss0, ss1, se0, se1):
    c = lax.axis_index("c")
    s = lax.axis_index("s")
    sem_src = [si0, si1]
    sem_dst = [sj0, sj1]
    sem_gth = [sg0, sg1]
    sem_rsc = [sr0, sr1]
    sem_ssc = [ss0, ss1]
    sem_exo = [se0, se1]

    pltpu.sync_copy(asad_h.at[pl.ds(2 * c * n, n)], asn_v)
    pltpu.sync_copy(asad_h.at[pl.ds((2 * c + 1) * n, n)], adn_v)

    for i in range(rpt // L):
      sbuf[pl.ds(i * L, L)] = jnp.zeros((L,), jnp.float32)
    pltpu.sync_copy(z2d_h.at[pl.ds(0, rpt)], acc.at[pl.ds(s * rpt, rpt)])
    pltpu.sync_copy(sbuf, s_sh.at[pl.ds(s * rpt, rpt)])

    @pl.when(s == NS - 1)
    def _():
      if tail:
        pltpu.sync_copy(z2d_h.at[pl.ds(0, tail)],
                        acc.at[pl.ds(NS * rpt, tail)])
        pltpu.sync_copy(sbuf.at[pl.ds(0, tail)],
                        s_sh.at[pl.ds(NS * rpt, tail)])

    plsc.subcore_barrier()

    base = s * ept

    def d_src(g, b):
      return pltpu.make_async_copy(
          src_h.at[pl.ds(base + g * CH, CH)], srcb.at[b], sem_src[b])

    def d_dst(g, b):
      return pltpu.make_async_copy(
          dst_h.at[pl.ds(base + g * CH, CH)], dstb.at[b], sem_dst[b])

    def d_gth(b):
      return pltpu.make_async_copy(
          tabs_h.at[srcp.at[b]], rows.at[b], sem_gth[b])

    def d_rsc(b):
      return pltpu.make_async_copy(
          rows.at[b], acc.at[dsc.at[b]], sem_rsc[b])

    def d_ssc(b):
      return pltpu.make_async_copy(
          exb.at[b], s_sh.at[dsc.at[b]], sem_ssc[b])

    def d_exo(g, b):
      return pltpu.make_async_copy(
          exb.at[b], ex_o.at[pl.ds(base + g * CH, CH)], sem_exo[b])

    def attn(b):
      for j in range(CH // L):
        sl = pl.ds(j * L, L)
        si = srcb[b, sl]
        di = dstb[b, sl]
        ev = plsc.load_gather(asn_v, [si]) + plsc.load_gather(adn_v, [di])
        ev = jnp.where(ev > 0, ev, 0.2 * ev)
        exb[b, sl] = jnp.exp(ev)
        srcp[b, sl] = si + c * n
        dsc[b, sl] = di

    def scale(b):
      for j in range(CH // L):
        grp = exb[b, pl.ds(j * L, L)]
        avs = [_bcast(grp, jj) for jj in range(L)]
        for jj in range(L):
          ei = j * L + jj
          for q in range(128 // L):
            sl = pl.ds(q * L, L)
            rows[b, ei, sl] = rows[b, ei, sl] * avs[jj]

    for b in range(2):
      d_src(b, b).start()
      d_dst(b, b).start()

    @pl.loop(0, nck2)
    def _it(g0):
      g = g0 * 2
      for b in range(2):
        d_src(g + b, b).wait()
        d_dst(g + b, b).wait()

        @pl.when(g0 >= 1)
        def _():
          d_rsc(b).wait()
          d_ssc(b).wait()

          @pl.when(c == 0)
          def _():
            d_exo(g + b - 2, b).wait()

        attn(b)
        d_gth(b).start()
        pltpu.async_copy(exb.at[b], s_sh.at[dsc.at[b]], sem_ssc[b], add=True)

        @pl.when(c == 0)
        def _():
          d_exo(g + b, b).start()

      @pl.when(g0 + 1 < nck2)
      def _():
        for b in range(2):
          d_src(g + 2 + b, b).start()
          d_dst(g + 2 + b, b).start()

      for b in range(2):
        d_gth(b).wait()
        scale(b)
        pltpu.async_copy(rows.at[b], acc.at[dsc.at[b]], sem_rsc[b], add=True)

    for b in range(2):
      d_rsc(b).wait()
      d_ssc(b).wait()

      @pl.when(c == 0)
      def _():
        d_exo(nck - 2 + b, b).wait()

    plsc.subcore_barrier()

    pltpu.sync_copy(acc.at[pl.ds(s * rpt, rpt)],
                    agg_o.at[pl.ds(c * n + s * rpt, rpt)])
    pltpu.sync_copy(s_sh.at[pl.ds(s * rpt, rpt)], sbuf)
    pltpu.sync_copy(sbuf, s_o.at[pl.ds(c * n + s * rpt, rpt)])

    @pl.when(s == NS - 1)
    def _():
      if tail:
        pltpu.sync_copy(acc.at[pl.ds(NS * rpt, tail)],
                        agg_o.at[pl.ds(c * n + NS * rpt, tail)])
        pltpu.sync_copy(s_sh.at[pl.ds(NS * rpt, tail)],
                        sbuf.at[pl.ds(0, tail)])
        pltpu.sync_copy(sbuf.at[pl.ds(0, tail)],
                        s_o.at[pl.ds(c * n + NS * rpt, tail)])

  return k(tabs2, asad, src, dst, z2d)


def _sc_conv2(h2c, zc, ex1, src, dst, z2d, n, e):
  ept = e // NS
  nck = ept // CH
  nck2 = nck // 2
  assert nck2 * 2 == nck
  rpt = (n // NS) // 8 * 8
  tail = n - rpt * NS

  mesh = plsc.VectorSubcoreMesh(core_axis_name="c", subcore_axis_name="s",
                                num_cores=NC, num_subcores=NS)

  @functools.partial(
      pl.kernel,
      mesh=mesh,
      compiler_params=pltpu.CompilerParams(needs_layout_passes=False),
      out_type=(
          jax.ShapeDtypeStruct((n, 128), jnp.float32),
          jax.ShapeDtypeStruct((n, 128), jnp.float32),
          jax.ShapeDtypeStruct((n,), jnp.float32),
      ),
      scratch_types=dict(
          srcb=pltpu.VMEM((2, CH), jnp.int32),
          dstb=pltpu.VMEM((2, CH), jnp.int32),
          ssc=pltpu.VMEM((2, CH), jnp.int32),
          dsc=pltpu.VMEM((2, CH), jnp.int32),
          exb=pltpu.VMEM((2, CH), jnp.float32),
          onesb=pltpu.VMEM((CH,), jnp.float32),
          rows=pltpu.VMEM((2, CH, 128), jnp.float32),
          sbuf=pltpu.VMEM((rpt,), jnp.float32),
          acc=pltpu.VMEM_SHARED((n, 128), jnp.float32),
          deg_sh=pltpu.VMEM_SHARED((n,), jnp.float32),
          si0=pltpu.SemaphoreType.DMA, si1=pltpu.SemaphoreType.DMA,
          sj0=pltpu.SemaphoreType.DMA, sj1=pltpu.SemaphoreType.DMA,
          sk0=pltpu.SemaphoreType.DMA, sk1=pltpu.SemaphoreType.DMA,
          sg0=pltpu.SemaphoreType.DMA, sg1=pltpu.SemaphoreType.DMA,
          sr0=pltpu.SemaphoreType.DMA, sr1=pltpu.SemaphoreType.DMA,
          sd0=pltpu.SemaphoreType.DMA, sd1=pltpu.SemaphoreType.DMA,
      ),
  )
  def k(h2_h, zc_h, ex_h, src_h, dst_h, z2d_h,
        aggb_o, vsum_o, deg_o,
        srcb, dstb, ssc, dsc, exb, onesb, rows, sbuf, acc, deg_sh,
        si0, si1, sj0, sj1, sk0, sk1, sg0, sg1, sr0, sr1, sd0, sd1):
    c = lax.axis_index("c")
    s = lax.axis_index("s")
    sem_src = [si0, si1]
    sem_dst = [sj0, sj1]
    sem_exl = [sk0, sk1]
    sem_gth = [sg0, sg1]
    sem_rsc = [sr0, sr1]
    sem_deg = [sd0, sd1]
    base = s * ept

    for i in range(rpt // L):
      sbuf[pl.ds(i * L, L)] = jnp.zeros((L,), jnp.float32)
    pltpu.sync_copy(z2d_h.at[pl.ds(0, rpt)], acc.at[pl.ds(s * rpt, rpt)])

    @pl.when(c == 0)
    def _():
      pltpu.sync_copy(sbuf, deg_sh.at[pl.ds(s * rpt, rpt)])

    @pl.when(s == NS - 1)
    def _():
      if tail:
        pltpu.sync_copy(z2d_h.at[pl.ds(0, tail)],
                        acc.at[pl.ds(NS * rpt, tail)])

        @pl.when(c == 0)
        def _():
          pltpu.sync_copy(sbuf.at[pl.ds(0, tail)],
                          deg_sh.at[pl.ds(NS * rpt, tail)])

    for j in range(CH // L):
      onesb[pl.ds(j * L, L)] = jnp.full((L,), 1.0, jnp.float32)

    plsc.subcore_barrier()

    def d_src(g, b):
      return pltpu.make_async_copy(
          src_h.at[pl.ds(base + g * CH, CH)], srcb.at[b], sem_src[b])

    def d_dst(g, b):
      return pltpu.make_async_copy(
          dst_h.at[pl.ds(base + g * CH, CH)], dstb.at[b], sem_dst[b])

    def d_exl(g, b):
      return pltpu.make_async_copy(
          ex_h.at[pl.ds(base + g * CH, CH)], exb.at[b], sem_exl[b])

    def d_gth0(b):
      return pltpu.make_async_copy(
          h2_h.at[ssc.at[b]], rows.at[b], sem_gth[b])

    def d_gth1(b):
      return pltpu.make_async_copy(
          zc_h.at[dsc.at[b]], rows.at[b], sem_gth[b])

    def d_rsc0(b):
      return pltpu.make_async_copy(
          rows.at[b], acc.at[dsc.at[b]], sem_rsc[b])

    def d_rsc1(b):
      return pltpu.make_async_copy(
          rows.at[b], acc.at[ssc.at[b]], sem_rsc[b])

    def d_deg(b):
      return pltpu.make_async_copy(
          onesb, deg_sh.at[ssc.at[b]], sem_deg[b])

    def copy_idx(b):
      for j in range(CH // L):
        sl = pl.ds(j * L, L)
        ssc[b, sl] = srcb[b, sl]
        dsc[b, sl] = dstb[b, sl]

    def scale64(b):
      for j in range(CH // L):
        grp = exb[b, pl.ds(j * L, L)]
        avs = [_bcast(grp, jj) for jj in range(L)]
        for jj in range(L):
          ei = j * L + jj
          for q in range(64 // L):
            sl = pl.ds(q * L, L)
            rows[b, ei, sl] = rows[b, ei, sl] * avs[jj]

    @pl.when(c == 0)
    def _core0():
      for b in range(2):
        d_src(b, b).start()
        d_dst(b, b).start()
        d_exl(b, b).start()

      @pl.loop(0, nck2)
      def _it(g0):
        g = g0 * 2
        for b in range(2):
          d_src(g + b, b).wait()
          d_dst(g + b, b).wait()

          @pl.when(g0 >= 1)
          def _():
            d_rsc0(b).wait()
            d_deg(b).wait()

          copy_idx(b)
          d_gth0(b).start()
          pltpu.async_copy(onesb, deg_sh.at[ssc.at[b]], sem_deg[b],
                           add=True)

        @pl.when(g0 + 1 < nck2)
        def _():
          for b in range(2):
            d_src(g + 2 + b, b).start()
            d_dst(g + 2 + b, b).start()

        for b in range(2):
          d_exl(g + b, b).wait()
          d_gth0(b).wait()
          scale64(b)
          pltpu.async_copy(rows.at[b], acc.at[dsc.at[b]], sem_rsc[b],
                           add=True)

        @pl.when(g0 + 1 < nck2)
        def _():
          for b in range(2):
            d_exl(g + 2 + b, b).start()

      for b in range(2):
        d_rsc0(b).wait()
        d_deg(b).wait()

    @pl.when(c == 1)
    def _core1():
      for b in range(2):
        d_src(b, b).start()
        d_dst(b, b).start()

      @pl.loop(0, nck2)
      def _it(g0):
        g = g0 * 2
        for b in range(2):
          d_src(g + b, b).wait()
          d_dst(g + b, b).wait()

          @pl.when(g0 >= 1)
          def _():
            d_rsc1(b).wait()

          copy_idx(b)
          d_gth1(b).start()

        @pl.when(g0 + 1 < nck2)
        def _():
          for b in range(2):
            d_src(g + 2 + b, b).start()
            d_dst(g + 2 + b, b).start()

        for b in range(2):
          d_gth1(b).wait()
          pltpu.async_copy(rows.at[b], acc.at[ssc.at[b]], sem_rsc[b],
                           add=True)

      for b in range(2):
        d_rsc1(b).wait()

    plsc.subcore_barrier()

    @pl.when(c == 0)
    def _dump0():
      pltpu.sync_copy(acc.at[pl.ds(s * rpt, rpt)],
                      aggb_o.at[pl.ds(s * rpt, rpt)])
      pltpu.sync_copy(deg_sh.at[pl.ds(s * rpt, rpt)], sbuf)
      pltpu.sync_copy(sbuf, deg_o.at[pl.ds(s * rpt, rpt)])

      @pl.when(s == NS - 1)
      def _():
        if tail:
          pltpu.sync_copy(acc.at[pl.ds(NS * rpt, tail)],
                          aggb_o.at[pl.ds(NS * rpt, tail)])
          pltpu.sync_copy(deg_sh.at[pl.ds(NS * rpt, tail)],
                          sbuf.at[pl.ds(0, tail)])
          pltpu.sync_copy(sbuf.at[pl.ds(0, tail)],
                          deg_o.at[pl.ds(NS * rpt, tail)])

    @pl.when(c == 1)
    def _dump1():
      pltpu.sync_copy(acc.at[pl.ds(s * rpt, rpt)],
                      vsum_o.at[pl.ds(s * rpt, rpt)])

      @pl.when(s == NS - 1)
      def _():
        if tail:
          pltpu.sync_copy(acc.at[pl.ds(NS * rpt, tail)],
                          vsum_o.at[pl.ds(NS * rpt, tail)])

  return k(h2c, zc, ex1, src, dst, z2d)


def _tc1(x, fa, W1, a2, n):
  def body(x_ref, fa_ref, w1_ref, a2_ref, out_ref):
    p = jnp.dot(w1_ref[...], a2_ref[...], preferred_element_type=jnp.float32)
    xa = jnp.dot(x_ref[...], p, preferred_element_type=jnp.float32)
    fb = jnp.dot(fa_ref[...], p, preferred_element_type=jnp.float32)
    out_ref[...] = jnp.concatenate([xa.T, fb.T], axis=0)

  return pl.pallas_call(
      body,
      out_shape=jax.ShapeDtypeStruct((4, n), jnp.float32),
  )(x, fa, W1, a2)


def _tc2(aggA, aggC, s1, sA, W1, W2, head1, n):
  grid = 5
  r = n // grid
  out = W2.shape[1]

  def body(aggA_ref, aggC_ref, s1_ref, sA_ref, w1_ref, w2_ref, hd_ref,
           h2_o, z_o, za_o, zc_o):
    inv1 = 1.0 / (s1_ref[...] + 1e-16)
    invA = 1.0 / (sA_ref[...] + 1e-16)
    h1 = _elu(jnp.dot(aggA_ref[...] * inv1, w1_ref[...],
                      preferred_element_type=jnp.float32))
    h2 = jnp.dot(h1, w2_ref[...], preferred_element_type=jnp.float32)
    z = jnp.dot(h2, hd_ref[...], preferred_element_type=jnp.float32)
    wza = jnp.dot(jnp.dot(w1_ref[...], w2_ref[...],
                          preferred_element_type=jnp.float32),
                  hd_ref[...], preferred_element_type=jnp.float32)
    za = jnp.dot(aggC_ref[...] * invA, wza,
                 preferred_element_type=jnp.float32)
    h2_o[...] = jnp.concatenate([h2, jnp.zeros_like(h2)], axis=1)
    z_o[...] = z
    za_o[...] = za
    zc_o[...] = jnp.concatenate([z, za], axis=1)

  return pl.pallas_call(
      body,
      grid=(grid,),
      in_specs=[
          pl.BlockSpec((r, aggA.shape[1]), lambda i: (i, 0)),
          pl.BlockSpec((r, aggC.shape[1]), lambda i: (i, 0)),
          pl.BlockSpec((r, 1), lambda i: (i, 0)),
          pl.BlockSpec((r, 1), lambda i: (i, 0)),
          pl.BlockSpec(W1.shape, lambda i: (0, 0)),
          pl.BlockSpec(W2.shape, lambda i: (0, 0)),
          pl.BlockSpec(head1.shape, lambda i: (0, 0)),
      ],
      out_specs=[
          pl.BlockSpec((r, 2 * out), lambda i: (i, 0)),
          pl.BlockSpec((r, out), lambda i: (i, 0)),
          pl.BlockSpec((r, out), lambda i: (i, 0)),
          pl.BlockSpec((r, 2 * out), lambda i: (i, 0)),
      ],
      out_shape=[
          jax.ShapeDtypeStruct((n, 2 * out), jnp.float32),
          jax.ShapeDtypeStruct((n, out), jnp.float32),
          jax.ShapeDtypeStruct((n, out), jnp.float32),
          jax.ShapeDtypeStruct((n, 2 * out), jnp.float32),
      ],
  )(aggA, aggC, s1, sA, W1, W2, head1)


def _tc3(aggB, s1, vsum2, deg, z, za, W1, W2, dW, db, n):
  grid = 5
  r = n // grid
  out = W2.shape[1]

  def body(aggB_ref, s1_ref, vs_ref, deg_ref, z_ref, za_ref,
           w1_ref, w2_ref, dw_ref, db_ref,
           h4_o, ret_o, reta_o, emb_o):
    inv1 = 1.0 / (s1_ref[...] + 1e-16)
    dn = (((1,), (1,)), ((), ()))
    aggb = aggB_ref[...][:, :w2_ref.shape[1]]
    h3 = _elu(lax.dot_general(aggb * inv1, w2_ref[...], dn,
                              preferred_element_type=jnp.float32))
    h4_o[...] = lax.dot_general(h3, w1_ref[...], dn,
                                preferred_element_type=jnp.float32)
    invd = 1.0 / deg_ref[...]
    vs = vs_ref[...]
    g1 = vs[:, :out] * invd
    g2 = vs[:, out:] * invd
    n1 = jnp.sqrt(jnp.sum(g1 * g1, axis=1, keepdims=True))
    n2 = jnp.sqrt(jnp.sum(g2 * g2, axis=1, keepdims=True))
    e1 = jax.nn.sigmoid(g1 / jnp.maximum(n1, 1e-12))
    e2 = jax.nn.sigmoid(g2 / jnp.maximum(n2, 1e-12))
    zw = jnp.dot(z_ref[...], dw_ref[...], preferred_element_type=jnp.float32)
    zaw = jnp.dot(za_ref[...], dw_ref[...], preferred_element_type=jnp.float32)
    b = db_ref[0, 0]
    ret_o[...] = jnp.concatenate(
        [jnp.sum(zw * e1, axis=1, keepdims=True) + b,
         jnp.sum(zaw * e1, axis=1, keepdims=True) + b], axis=1)
    reta_o[...] = jnp.concatenate(
        [jnp.sum(zaw * e2, axis=1, keepdims=True) + b,
         jnp.sum(zw * e2, axis=1, keepdims=True) + b], axis=1)
    emb_o[...] = e1

  return pl.pallas_call(
      body,
      grid=(grid,),
      in_specs=[
          pl.BlockSpec((r, aggB.shape[1]), lambda i: (i, 0)),
          pl.BlockSpec((r, 1), lambda i: (i, 0)),
          pl.BlockSpec((r, vsum2.shape[1]), lambda i: (i, 0)),
          pl.BlockSpec((r, 1), lambda i: (i, 0)),
          pl.BlockSpec((r, out), lambda i: (i, 0)),
          pl.BlockSpec((r, out), lambda i: (i, 0)),
          pl.BlockSpec(W1.shape, lambda i: (0, 0)),
          pl.BlockSpec(W2.shape, lambda i: (0, 0)),
          pl.BlockSpec(dW.shape, lambda i: (0, 0)),
          pl.BlockSpec((1, 1), lambda i: (0, 0)),
      ],
      out_specs=[
          pl.BlockSpec((r, W1.shape[0]), lambda i: (i, 0)),
          pl.BlockSpec((r, 2), lambda i: (i, 0)),
          pl.BlockSpec((r, 2), lambda i: (i, 0)),
          pl.BlockSpec((r, out), lambda i: (i, 0)),
      ],
      out_shape=[
          jax.ShapeDtypeStruct((n, W1.shape[0]), jnp.float32),
          jax.ShapeDtypeStruct((n, 2), jnp.float32),
          jax.ShapeDtypeStruct((n, 2), jnp.float32),
          jax.ShapeDtypeStruct((n, out), jnp.float32),
      ],
  )(aggB, s1, vsum2, deg, z, za, W1, W2, dW, db)


def kernel(features, edge_index, feat_a, W1, W2, att_src, att_dst, head1,
           disc_W, disc_b):
  n = features.shape[0]
  e = edge_index.shape[1]
  assert n % NS == 0 and e % (NS * CH) == 0

  src = edge_index[0]
  dst = edge_index[1]

  a2 = jnp.stack([att_src, att_dst], axis=1)
  asad = _tc1(features, feat_a, W1, a2, n).reshape(-1)

  tabs2 = jnp.concatenate([features, feat_a], axis=0)
  z2d = jnp.zeros((656, 128), jnp.float32)

  agg, s2, ex1 = _sc_conv1(tabs2, asad, src, dst, z2d, n, e)
  aggA, aggC = agg[:n], agg[n:]
  s1 = s2[:n].reshape(n, 1)
  sA = s2[n:].reshape(n, 1)

  h2c, z, za, zc = _tc2(aggA, aggC, s1, sA, W1, W2, head1, n)

  aggB, vsum2, deg = _sc_conv2(h2c, zc, ex1, src, dst, z2d, n, e)

  h4, ret, ret_a, emb = _tc3(aggB, s1, vsum2, deg.reshape(n, 1), z, za,
                             W1, W2, disc_W, disc_b.reshape(1, 1), n)

  return (z, h4, ret, ret_a, emb)


# speedup vs baseline: 1.0367x; 1.0367x over previous
import functools

import jax
import jax.numpy as jnp
from jax import lax
from jax.experimental import pallas as pl
from jax.experimental.pallas import tpu as pltpu
from jax.experimental.pallas import tpu_sc as plsc

NC = 2
NS = 16
L = 16
CH = 32
NB = 5


def _elu(x):
  return jnp.where(x > 0, x, jnp.exp(x) - 1.0)


def _bcast(grp, j):
  v = jnp.sum(jnp.where(jnp.arange(L, dtype=jnp.int32) == j, grp, 0.0))
  return jnp.broadcast_to(v, (L,))


def _sc_conv1(tabs2, asad, src, dst, z2d, n, e):
  ept = e // NS
  nck = ept // CH
  nckr = nck // NB
  assert nckr * NB == nck
  rpt = (n // NS) // 8 * 8
  tail = n - rpt * NS

  mesh = plsc.VectorSubcoreMesh(core_axis_name="c", subcore_axis_name="s",
                                num_cores=NC, num_subcores=NS)

  @functools.partial(
      pl.kernel,
      mesh=mesh,
      compiler_params=pltpu.CompilerParams(needs_layout_passes=False),
      out_type=(
          jax.ShapeDtypeStruct((2 * n, 128), jnp.float32),
          jax.ShapeDtypeStruct((2 * n,), jnp.float32),
          jax.ShapeDtypeStruct((e,), jnp.float32),
      ),
      scratch_types=dict(
          asn_v=pltpu.VMEM((n,), jnp.float32),
          adn_v=pltpu.VMEM((n,), jnp.float32),
          srcb=pltpu.VMEM((NB, CH), jnp.int32),
          dstb=pltpu.VMEM((NB, CH), jnp.int32),
          srcp=pltpu.VMEM((NB, CH), jnp.int32),
          dsc=pltpu.VMEM((NB, CH), jnp.int32),
          exb=pltpu.VMEM((NB, CH), jnp.float32),
          rows0=pltpu.VMEM((CH, 128), jnp.float32),
          rows1=pltpu.VMEM((CH, 128), jnp.float32),
          rows2=pltpu.VMEM((CH, 128), jnp.float32),
          rows3=pltpu.VMEM((CH, 128), jnp.float32),
          rows4=pltpu.VMEM((CH, 128), jnp.float32),
          sbuf=pltpu.VMEM((rpt,), jnp.float32),
          acc=pltpu.VMEM_SHARED((n, 128), jnp.float32),
          s_sh=pltpu.VMEM_SHARED((n,), jnp.float32),
          sem_i=pltpu.SemaphoreType.DMA((NB,)),
          sem_j=pltpu.SemaphoreType.DMA((NB,)),
          sem_g=pltpu.SemaphoreType.DMA((NB,)),
          sem_r=pltpu.SemaphoreType.DMA((NB,)),
          sem_s=pltpu.SemaphoreType.DMA((NB,)),
          sem_e=pltpu.SemaphoreType.DMA((NB,)),
      ),
  )
  def k(tabs_h, asad_h, src_h, dst_h, z2d_h,
        agg_o, s_o, ex_o,
        asn_v, adn_v, srcb, dstb, srcp, dsc, exb,
        rows0, rows1, rows2, rows3, rows4, sbuf, acc, s_sh,
        sem_i, sem_j, sem_g, sem_r, sem_s, sem_e):
    c = lax.axis_index("c")
    s = lax.axis_index("s")
    rowsl = [rows0, rows1, rows2, rows3, rows4]
    sem_src = [sem_i.at[b] for b in range(NB)]
    sem_dst = [sem_j.at[b] for b in range(NB)]
    sem_gth = [sem_g.at[b] for b in range(NB)]
    sem_rsc = [sem_r.at[b] for b in range(NB)]
    sem_ssc = [sem_s.at[b] for b in range(NB)]
    sem_exo = [sem_e.at[b] for b in range(NB)]

    pltpu.sync_copy(asad_h.at[pl.ds(2 * c * n, n)], asn_v)
    pltpu.sync_copy(asad_h.at[pl.ds((2 * c + 1) * n, n)], adn_v)

    for i in range(rpt // L):
      sbuf[pl.ds(i * L, L)] = jnp.zeros((L,), jnp.float32)
    pltpu.sync_copy(z2d_h.at[pl.ds(0, rpt)], acc.at[pl.ds(s * rpt, rpt)])
    pltpu.sync_copy(sbuf, s_sh.at[pl.ds(s * rpt, rpt)])

    @pl.when(s == NS - 1)
    def _():
      if tail:
        pltpu.sync_copy(z2d_h.at[pl.ds(0, tail)],
                        acc.at[pl.ds(NS * rpt, tail)])
        pltpu.sync_copy(sbuf.at[pl.ds(0, tail)],
                        s_sh.at[pl.ds(NS * rpt, tail)])

    plsc.subcore_barrier()

    base = s * ept

    def d_src(g, b):
      return pltpu.make_async_copy(
          src_h.at[pl.ds(base + g * CH, CH)], srcb.at[b], sem_src[b])

    def d_dst(g, b):
      return pltpu.make_async_copy(
          dst_h.at[pl.ds(base + g * CH, CH)], dstb.at[b], sem_dst[b])

    def d_gth(b):
      return pltpu.make_async_copy(
          tabs_h.at[srcp.at[b]], rowsl[b], sem_gth[b])

    def d_rsc(b):
      return pltpu.make_async_copy(
          rowsl[b], acc.at[dsc.at[b]], sem_rsc[b])

    def d_ssc(b):
      return pltpu.make_async_copy(
          exb.at[b], s_sh.at[dsc.at[b]], sem_ssc[b])

    def d_exo(g, b):
      return pltpu.make_async_copy(
          exb.at[b], ex_o.at[pl.ds(base + g * CH, CH)], sem_exo[b])

    def attn(b):
      for j in range(CH // L):
        sl = pl.ds(j * L, L)
        si = srcb[b, sl]
        di = dstb[b, sl]
        ev = plsc.load_gather(asn_v, [si]) + plsc.load_gather(adn_v, [di])
        ev = jnp.where(ev > 0, ev, 0.2 * ev)
        exb[b, sl] = jnp.exp(ev)
        srcp[b, sl] = si + c * n
        dsc[b, sl] = di

    def scale(b):
      for j in range(CH // L):
        grp = exb[b, pl.ds(j * L, L)]
        avs = [_bcast(grp, jj) for jj in range(L)]
        for jj in range(L):
          ei = j * L + jj
          for q in range(128 // L):
            sl = pl.ds(q * L, L)
            rowsl[b][ei, sl] = rowsl[b][ei, sl] * avs[jj]

    for b in range(NB):
      d_src(b, b).start()
      d_dst(b, b).start()

    @pl.loop(0, nckr)
    def _it(g0):
      g = g0 * NB
      for b in range(NB):
        d_src(g + b, b).wait()
        d_dst(g + b, b).wait()

        @pl.when(g0 >= 1)
        def _():
          d_rsc(b).wait()
          d_ssc(b).wait()

          @pl.when(c == 0)
          def _():
            d_exo(g + b - NB, b).wait()

        attn(b)
        d_gth(b).start()
        pltpu.async_copy(exb.at[b], s_sh.at[dsc.at[b]], sem_ssc[b], add=True)

        @pl.when(c == 0)
        def _():
          d_exo(g + b, b).start()

      @pl.when(g0 + 1 < nckr)
      def _():
        for b in range(NB):
          d_src(g + NB + b, b).start()
          d_dst(g + NB + b, b).start()

      for b in range(NB):
        d_gth(b).wait()
        scale(b)
        pltpu.async_copy(rowsl[b], acc.at[dsc.at[b]], sem_rsc[b], add=True)

    for b in range(NB):
      d_rsc(b).wait()
      d_ssc(b).wait()

      @pl.when(c == 0)
      def _():
        d_exo(nck - NB + b, b).wait()

    plsc.subcore_barrier()

    pltpu.sync_copy(acc.at[pl.ds(s * rpt, rpt)],
                    agg_o.at[pl.ds(c * n + s * rpt, rpt)])
    pltpu.sync_copy(s_sh.at[pl.ds(s * rpt, rpt)], sbuf)
    pltpu.sync_copy(sbuf, s_o.at[pl.ds(c * n + s * rpt, rpt)])

    @pl.when(s == NS - 1)
    def _():
      if tail:
        pltpu.sync_copy(acc.at[pl.ds(NS * rpt, tail)],
                        agg_o.at[pl.ds(c * n + NS * rpt, tail)])
        pltpu.sync_copy(s_sh.at[pl.ds(NS * rpt, tail)],
                        sbuf.at[pl.ds(0, tail)])
        pltpu.sync_copy(sbuf.at[pl.ds(0, tail)],
                        s_o.at[pl.ds(c * n + NS * rpt, tail)])

  return k(tabs2, asad, src, dst, z2d)


def _sc_conv2(h2c, zc, ex1, src, dst, z2d, n, e):
  ept = e // NS
  nck = ept // CH
  nckr = nck // NB
  assert nckr * NB == nck
  rpt = (n // NS) // 8 * 8
  tail = n - rpt * NS

  mesh = plsc.VectorSubcoreMesh(core_axis_name="c", subcore_axis_name="s",
                                num_cores=NC, num_subcores=NS)

  @functools.partial(
      pl.kernel,
      mesh=mesh,
      compiler_params=pltpu.CompilerParams(needs_layout_passes=False),
      out_type=(
          jax.ShapeDtypeStruct((n, 128), jnp.float32),
          jax.ShapeDtypeStruct((n, 128), jnp.float32),
          jax.ShapeDtypeStruct((n,), jnp.float32),
      ),
      scratch_types=dict(
          srcb=pltpu.VMEM((NB, CH), jnp.int32),
          dstb=pltpu.VMEM((NB, CH), jnp.int32),
          ssc=pltpu.VMEM((NB, CH), jnp.int32),
          dsc=pltpu.VMEM((NB, CH), jnp.int32),
          exb=pltpu.VMEM((NB, CH), jnp.float32),
          onesb=pltpu.VMEM((CH,), jnp.float32),
          rows0=pltpu.VMEM((CH, 128), jnp.float32),
          rows1=pltpu.VMEM((CH, 128), jnp.float32),
          rows2=pltpu.VMEM((CH, 128), jnp.float32),
          rows3=pltpu.VMEM((CH, 128), jnp.float32),
          rows4=pltpu.VMEM((CH, 128), jnp.float32),
          sbuf=pltpu.VMEM((rpt,), jnp.float32),
          acc=pltpu.VMEM_SHARED((n, 128), jnp.float32),
          deg_sh=pltpu.VMEM_SHARED((n,), jnp.float32),
          sem_i=pltpu.SemaphoreType.DMA((NB,)),
          sem_j=pltpu.SemaphoreType.DMA((NB,)),
          sem_k=pltpu.SemaphoreType.DMA((NB,)),
          sem_g=pltpu.SemaphoreType.DMA((NB,)),
          sem_r=pltpu.SemaphoreType.DMA((NB,)),
          sem_d=pltpu.SemaphoreType.DMA((NB,)),
      ),
  )
  def k(h2_h, zc_h, ex_h, src_h, dst_h, z2d_h,
        aggb_o, vsum_o, deg_o,
        srcb, dstb, ssc, dsc, exb, onesb,
        rows0, rows1, rows2, rows3, rows4, sbuf, acc, deg_sh,
        sem_i, sem_j, sem_k, sem_g, sem_r, sem_d):
    c = lax.axis_index("c")
    s = lax.axis_index("s")
    rowsl = [rows0, rows1, rows2, rows3, rows4]
    sem_src = [sem_i.at[b] for b in range(NB)]
    sem_dst = [sem_j.at[b] for b in range(NB)]
    sem_exl = [sem_k.at[b] for b in range(NB)]
    sem_gth = [sem_g.at[b] for b in range(NB)]
    sem_rsc = [sem_r.at[b] for b in range(NB)]
    sem_deg = [sem_d.at[b] for b in range(NB)]
    base = s * ept

    for i in range(rpt // L):
      sbuf[pl.ds(i * L, L)] = jnp.zeros((L,), jnp.float32)
    pltpu.sync_copy(z2d_h.at[pl.ds(0, rpt)], acc.at[pl.ds(s * rpt, rpt)])

    @pl.when(c == 0)
    def _():
      pltpu.sync_copy(sbuf, deg_sh.at[pl.ds(s * rpt, rpt)])

    @pl.when(s == NS - 1)
    def _():
      if tail:
        pltpu.sync_copy(z2d_h.at[pl.ds(0, tail)],
                        acc.at[pl.ds(NS * rpt, tail)])

        @pl.when(c == 0)
        def _():
          pltpu.sync_copy(sbuf.at[pl.ds(0, tail)],
                          deg_sh.at[pl.ds(NS * rpt, tail)])

    for j in range(CH // L):
      onesb[pl.ds(j * L, L)] = jnp.full((L,), 1.0, jnp.float32)

    plsc.subcore_barrier()

    def d_src(g, b):
      return pltpu.make_async_copy(
          src_h.at[pl.ds(base + g * CH, CH)], srcb.at[b], sem_src[b])

    def d_dst(g, b):
      return pltpu.make_async_copy(
          dst_h.at[pl.ds(base + g * CH, CH)], dstb.at[b], sem_dst[b])

    def d_exl(g, b):
      return pltpu.make_async_copy(
          ex_h.at[pl.ds(base + g * CH, CH)], exb.at[b], sem_exl[b])

    def d_gth0(b):
      return pltpu.make_async_copy(
          h2_h.at[ssc.at[b]], rowsl[b], sem_gth[b])

    def d_gth1(b):
      return pltpu.make_async_copy(
          zc_h.at[dsc.at[b]], rowsl[b], sem_gth[b])

    def d_rsc0(b):
      return pltpu.make_async_copy(
          rowsl[b], acc.at[dsc.at[b]], sem_rsc[b])

    def d_rsc1(b):
      return pltpu.make_async_copy(
          rowsl[b], acc.at[ssc.at[b]], sem_rsc[b])

    def d_deg(b):
      return pltpu.make_async_copy(
          onesb, deg_sh.at[ssc.at[b]], sem_deg[b])

    def copy_idx(b):
      for j in range(CH // L):
        sl = pl.ds(j * L, L)
        ssc[b, sl] = srcb[b, sl]
        dsc[b, sl] = dstb[b, sl]

    def scale64(b):
      for j in range(CH // L):
        grp = exb[b, pl.ds(j * L, L)]
        avs = [_bcast(grp, jj) for jj in range(L)]
        for jj in range(L):
          ei = j * L + jj
          for q in range(64 // L):
            sl = pl.ds(q * L, L)
            rowsl[b][ei, sl] = rowsl[b][ei, sl] * avs[jj]

    @pl.when(c == 0)
    def _core0():
      for b in range(NB):
        d_src(b, b).start()
        d_dst(b, b).start()
        d_exl(b, b).start()

      @pl.loop(0, nckr)
      def _it(g0):
        g = g0 * NB
        for b in range(NB):
          d_src(g + b, b).wait()
          d_dst(g + b, b).wait()

          @pl.when(g0 >= 1)
          def _():
            d_rsc0(b).wait()
            d_deg(b).wait()

          copy_idx(b)
          d_gth0(b).start()
          pltpu.async_copy(onesb, deg_sh.at[ssc.at[b]], sem_deg[b],
                           add=True)

        @pl.when(g0 + 1 < nckr)
        def _():
          for b in range(NB):
            d_src(g + NB + b, b).start()
            d_dst(g + NB + b, b).start()

        for b in range(NB):
          d_exl(g + b, b).wait()
          d_gth0(b).wait()
          scale64(b)
          pltpu.async_copy(rowsl[b], acc.at[dsc.at[b]], sem_rsc[b],
                           add=True)

        @pl.when(g0 + 1 < nckr)
        def _():
          for b in range(NB):
            d_exl(g + NB + b, b).start()

      for b in range(NB):
        d_rsc0(b).wait()
        d_deg(b).wait()

    @pl.when(c == 1)
    def _core1():
      for b in range(NB):
        d_src(b, b).start()
        d_dst(b, b).start()

      @pl.loop(0, nckr)
      def _it(g0):
        g = g0 * NB
        for b in range(NB):
          d_src(g + b, b).wait()
          d_dst(g + b, b).wait()

          @pl.when(g0 >= 1)
          def _():
            d_rsc1(b).wait()

          copy_idx(b)
          d_gth1(b).start()

        @pl.when(g0 + 1 < nckr)
        def _():
          for b in range(NB):
            d_src(g + NB + b, b).start()
            d_dst(g + NB + b, b).start()

        for b in range(NB):
          d_gth1(b).wait()
          pltpu.async_copy(rowsl[b], acc.at[ssc.at[b]], sem_rsc[b],
                           add=True)

      for b in range(NB):
        d_rsc1(b).wait()

    plsc.subcore_barrier()

    @pl.when(c == 0)
    def _dump0():
      pltpu.sync_copy(acc.at[pl.ds(s * rpt, rpt)],
                      aggb_o.at[pl.ds(s * rpt, rpt)])
      pltpu.sync_copy(deg_sh.at[pl.ds(s * rpt, rpt)], sbuf)
      pltpu.sync_copy(sbuf, deg_o.at[pl.ds(s * rpt, rpt)])

      @pl.when(s == NS - 1)
      def _():
        if tail:
          pltpu.sync_copy(acc.at[pl.ds(NS * rpt, tail)],
                          aggb_o.at[pl.ds(NS * rpt, tail)])
          pltpu.sync_copy(deg_sh.at[pl.ds(NS * rpt, tail)],
                          sbuf.at[pl.ds(0, tail)])
          pltpu.sync_copy(sbuf.at[pl.ds(0, tail)],
                          deg_o.at[pl.ds(NS * rpt, tail)])

    @pl.when(c == 1)
    def _dump1():
      pltpu.sync_copy(acc.at[pl.ds(s * rpt, rpt)],
                      vsum_o.at[pl.ds(s * rpt, rpt)])

      @pl.when(s == NS - 1)
      def _():
        if tail:
          pltpu.sync_copy(acc.at[pl.ds(NS * rpt, tail)],
                          vsum_o.at[pl.ds(NS * rpt, tail)])

  return k(h2c, zc, ex1, src, dst, z2d)


def _tc1(x, fa, W1, a2, n):
  def body(x_ref, fa_ref, w1_ref, a2_ref, out_ref):
    p = jnp.dot(w1_ref[...], a2_ref[...], preferred_element_type=jnp.float32)
    xa = jnp.dot(x_ref[...], p, preferred_element_type=jnp.float32)
    fb = jnp.dot(fa_ref[...], p, preferred_element_type=jnp.float32)
    out_ref[...] = jnp.concatenate([xa.T, fb.T], axis=0)

  return pl.pallas_call(
      body,
      out_shape=jax.ShapeDtypeStruct((4, n), jnp.float32),
  )(x, fa, W1, a2)


def _tc2(aggA, aggC, s1, sA, W1, W2, head1, n):
  grid = 5
  r = n // grid
  out = W2.shape[1]

  def body(aggA_ref, aggC_ref, s1_ref, sA_ref, w1_ref, w2_ref, hd_ref,
           h2_o, z_o, za_o, zc_o):
    inv1 = 1.0 / (s1_ref[...] + 1e-16)
    invA = 1.0 / (sA_ref[...] + 1e-16)
    h1 = _elu(jnp.dot(aggA_ref[...] * inv1, w1_ref[...],
                      preferred_element_type=jnp.float32))
    h2 = jnp.dot(h1, w2_ref[...], preferred_element_type=jnp.float32)
    z = jnp.dot(h2, hd_ref[...], preferred_element_type=jnp.float32)
    wza = jnp.dot(jnp.dot(w1_ref[...], w2_ref[...],
                          preferred_element_type=jnp.float32),
                  hd_ref[...], preferred_element_type=jnp.float32)
    za = jnp.dot(aggC_ref[...] * invA, wza,
                 preferred_element_type=jnp.float32)
    h2_o[...] = jnp.concatenate([h2, jnp.zeros_like(h2)], axis=1)
    z_o[...] = z
    za_o[...] = za
    zc_o[...] = jnp.concatenate([z, za], axis=1)

  return pl.pallas_call(
      body,
      grid=(grid,),
      in_specs=[
          pl.BlockSpec((r, aggA.shape[1]), lambda i: (i, 0)),
          pl.BlockSpec((r, aggC.shape[1]), lambda i: (i, 0)),
          pl.BlockSpec((r, 1), lambda i: (i, 0)),
          pl.BlockSpec((r, 1), lambda i: (i, 0)),
          pl.BlockSpec(W1.shape, lambda i: (0, 0)),
          pl.BlockSpec(W2.shape, lambda i: (0, 0)),
          pl.BlockSpec(head1.shape, lambda i: (0, 0)),
      ],
      out_specs=[
          pl.BlockSpec((r, 2 * out), lambda i: (i, 0)),
          pl.BlockSpec((r, out), lambda i: (i, 0)),
          pl.BlockSpec((r, out), lambda i: (i, 0)),
          pl.BlockSpec((r, 2 * out), lambda i: (i, 0)),
      ],
      out_shape=[
          jax.ShapeDtypeStruct((n, 2 * out), jnp.float32),
          jax.ShapeDtypeStruct((n, out), jnp.float32),
          jax.ShapeDtypeStruct((n, out), jnp.float32),
          jax.ShapeDtypeStruct((n, 2 * out), jnp.float32),
      ],
  )(aggA, aggC, s1, sA, W1, W2, head1)


def _tc3(aggB, s1, vsum2, deg, z, za, W1, W2, dW, db, n):
  grid = 5
  r = n // grid
  out = W2.shape[1]

  def body(aggB_ref, s1_ref, vs_ref, deg_ref, z_ref, za_ref,
           w1_ref, w2_ref, dw_ref, db_ref,
           h4_o, ret_o, reta_o, emb_o):
    inv1 = 1.0 / (s1_ref[...] + 1e-16)
    dn = (((1,), (1,)), ((), ()))
    aggb = aggB_ref[...][:, :w2_ref.shape[1]]
    h3 = _elu(lax.dot_general(aggb * inv1, w2_ref[...], dn,
                              preferred_element_type=jnp.float32))
    h4_o[...] = lax.dot_general(h3, w1_ref[...], dn,
                                preferred_element_type=jnp.float32)
    invd = 1.0 / deg_ref[...]
    vs = vs_ref[...]
    g1 = vs[:, :out] * invd
    g2 = vs[:, out:] * invd
    n1 = jnp.sqrt(jnp.sum(g1 * g1, axis=1, keepdims=True))
    n2 = jnp.sqrt(jnp.sum(g2 * g2, axis=1, keepdims=True))
    e1 = jax.nn.sigmoid(g1 / jnp.maximum(n1, 1e-12))
    e2 = jax.nn.sigmoid(g2 / jnp.maximum(n2, 1e-12))
    zw = jnp.dot(z_ref[...], dw_ref[...], preferred_element_type=jnp.float32)
    zaw = jnp.dot(za_ref[...], dw_ref[...], preferred_element_type=jnp.float32)
    b = db_ref[0, 0]
    ret_o[...] = jnp.concatenate(
        [jnp.sum(zw * e1, axis=1, keepdims=True) + b,
         jnp.sum(zaw * e1, axis=1, keepdims=True) + b], axis=1)
    reta_o[...] = jnp.concatenate(
        [jnp.sum(zaw * e2, axis=1, keepdims=True) + b,
         jnp.sum(zw * e2, axis=1, keepdims=True) + b], axis=1)
    emb_o[...] = e1

  return pl.pallas_call(
      body,
      grid=(grid,),
      in_specs=[
          pl.BlockSpec((r, aggB.shape[1]), lambda i: (i, 0)),
          pl.BlockSpec((r, 1), lambda i: (i, 0)),
          pl.BlockSpec((r, vsum2.shape[1]), lambda i: (i, 0)),
          pl.BlockSpec((r, 1), lambda i: (i, 0)),
          pl.BlockSpec((r, out), lambda i: (i, 0)),
          pl.BlockSpec((r, out), lambda i: (i, 0)),
          pl.BlockSpec(W1.shape, lambda i: (0, 0)),
          pl.BlockSpec(W2.shape, lambda i: (0, 0)),
          pl.BlockSpec(dW.shape, lambda i: (0, 0)),
          pl.BlockSpec((1, 1), lambda i: (0, 0)),
      ],
      out_specs=[
          pl.BlockSpec((r, W1.shape[0]), lambda i: (i, 0)),
          pl.BlockSpec((r, 2), lambda i: (i, 0)),
          pl.BlockSpec((r, 2), lambda i: (i, 0)),
          pl.BlockSpec((r, out), lambda i: (i, 0)),
      ],
      out_shape=[
          jax.ShapeDtypeStruct((n, W1.shape[0]), jnp.float32),
          jax.ShapeDtypeStruct((n, 2), jnp.float32),
          jax.ShapeDtypeStruct((n, 2), jnp.float32),
          jax.ShapeDtypeStruct((n, out), jnp.float32),
      ],
  )(aggB, s1, vsum2, deg, z, za, W1, W2, dW, db)


def kernel(features, edge_index, feat_a, W1, W2, att_src, att_dst, head1,
           disc_W, disc_b):
  n = features.shape[0]
  e = edge_index.shape[1]
  assert n % NS == 0 and e % (NS * CH) == 0

  src = edge_index[0]
  dst = edge_index[1]

  a2 = jnp.stack([att_src, att_dst], axis=1)
  asad = _tc1(features, feat_a, W1, a2, n).reshape(-1)

  tabs2 = jnp.concatenate([features, feat_a], axis=0)
  z2d = jnp.zeros((656, 128), jnp.float32)

  agg, s2, ex1 = _sc_conv1(tabs2, asad, src, dst, z2d, n, e)
  aggA, aggC = agg[:n], agg[n:]
  s1 = s2[:n].reshape(n, 1)
  sA = s2[n:].reshape(n, 1)

  h2c, z, za, zc = _tc2(aggA, aggC, s1, sA, W1, W2, head1, n)

  aggB, vsum2, deg = _sc_conv2(h2c, zc, ex1, src, dst, z2d, n, e)

  h4, ret, ret_a, emb = _tc3(aggB, s1, vsum2, deg.reshape(n, 1), z, za,
                             W1, W2, disc_W, disc_b.reshape(1, 1), n)

  return (z, h4, ret, ret_a, emb)
